# baseline (speedup 1.0000x reference)
import functools

import jax
import jax.numpy as jnp
import numpy as np
from jax import lax
from jax.experimental import pallas as pl
from jax.experimental.pallas import tpu as pltpu
from jax.experimental.pallas import tpu_sc as plsc

B = 16
L = 200
LA = 10
D_IN = 1024
D = 128
P = 75
S_OUT = 1 + LA + 2 + L + P

_HALF = L // 2
_WLOAD = 104
_PSPLIT = 40
_ASPLIT = 8
_NROWS = _WLOAD + _PSPLIT + _ASPLIT
_ACHUNK = 16
_IDXROW = 160


def _dst_tables():
    base = np.arange(B, dtype=np.int32)[:, None] * S_OUT
    h0 = np.concatenate([[0], 13 + np.arange(_HALF), [112] * (_WLOAD - _HALF - 1)])
    h1 = np.concatenate([113 + np.arange(_HALF), [212] * (_WLOAD - _HALF)])
    pos = np.concatenate([213 + np.arange(P), [287] * (2 * _PSPLIT - P)])
    arow = np.concatenate([1 + np.arange(12), [12] * (2 * _ASPLIT - 12)])
    c0 = np.concatenate([h0, pos[:_PSPLIT], arow[:_ASPLIT]]).astype(np.int32)
    c1 = np.concatenate([h1, pos[_PSPLIT:], arow[_ASPLIT:]]).astype(np.int32)
    return np.concatenate([(c0[None, :] + base).reshape(-1),
                           (c1[None, :] + base).reshape(-1)])


_DST = _dst_tables()


def _tc_body(ca_ref, sep_ref, pt_ref, abs_ref, w_ref, tw_ref,
             absw_ref, mask_ref, idx_ref):
    a = abs_ref[...].reshape(B * LA, D_IN)
    proj = lax.dot_general(a, w_ref[...], (((1,), (0,)), ((), ())),
                           preferred_element_type=jnp.float32)
    tw = tw_ref[...]
    t0 = jnp.broadcast_to(tw[0][None, None, :], (B, 1, D))
    t1 = jnp.broadcast_to(tw[1][None, None, :], (B, 5, D))
    absw_ref[...] = jnp.concatenate([proj.reshape(B, LA, D), t0, t1], axis=1)
    mask_ref[...] = jnp.ones((B, S_OUT), jnp.float32)
    ca = ca_ref[...]
    pt = pt_ref[...]
    h0 = jnp.concatenate(
        [sep_ref[...][:, :1], ca[:, :_HALF],
         jnp.broadcast_to(ca[:, _HALF - 1:_HALF], (B, _WLOAD - _HALF - 1)),
         pt[:, :_PSPLIT],
         jnp.broadcast_to(pt[:, P - 1:P], (B, _IDXROW - _WLOAD - _PSPLIT))], axis=1)
    h1 = jnp.concatenate(
        [ca[:, _HALF:], jnp.broadcast_to(ca[:, L - 1:L], (B, _WLOAD - _HALF)),
         pt[:, _PSPLIT:],
         jnp.broadcast_to(pt[:, P - 1:P], (B, _IDXROW - _WLOAD - P + _PSPLIT))], axis=1)
    idx_ref[...] = jnp.concatenate([h0, h1], axis=0)


_tc_call = pl.pallas_call(
    _tc_body,
    out_shape=(
        jax.ShapeDtypeStruct((B, _ACHUNK, D), jnp.float32),
        jax.ShapeDtypeStruct((B, S_OUT), jnp.float32),
        jax.ShapeDtypeStruct((2 * B, _IDXROW), jnp.int32),
    ),
)


def _sc_body(wte, pos_table, absw, idx, dst, out,
             idx_v, dst_v, rows_v, sem0, sem1, sem2):
    c = lax.axis_index("c")
    s = lax.axis_index("s")
    b = s
    chunk = c * B + s

    a1 = pltpu.async_copy(absw.at[b, pl.ds(c * _ASPLIT, _ASPLIT)],
                          rows_v.at[pl.ds(_WLOAD + _PSPLIT, _ASPLIT)], sem2)
    l1 = pltpu.async_copy(idx.at[pl.ds(chunk * _IDXROW, _IDXROW)], idx_v, sem0)
    l3 = pltpu.async_copy(dst.at[pl.ds(chunk * _NROWS, _NROWS)], dst_v, sem1)
    l1.wait()
    g1 = pltpu.async_copy(wte.at[idx_v.at[pl.ds(0, _WLOAD)]],
                          rows_v.at[pl.ds(0, _WLOAD)], sem0)
    g2 = pltpu.async_copy(pos_table.at[idx_v.at[pl.ds(_WLOAD, _PSPLIT)]],
                          rows_v.at[pl.ds(_WLOAD, _PSPLIT)], sem0)
    l3.wait()
    a1.wait()
    g1.wait()
    g2.wait()
    pltpu.async_copy(rows_v, out.at[dst_v], sem2).wait()


_sc_call = functools.partial(
    pl.kernel,
    out_type=jax.ShapeDtypeStruct((B * S_OUT, D), jnp.float32),
    mesh=plsc.VectorSubcoreMesh(core_axis_name="c", subcore_axis_name="s"),
    scratch_types=[
        pltpu.VMEM((_IDXROW,), jnp.int32),
        pltpu.VMEM((_NROWS,), jnp.int32),
        pltpu.VMEM((_NROWS, D), jnp.float32),
        pltpu.SemaphoreType.DMA,
        pltpu.SemaphoreType.DMA,
        pltpu.SemaphoreType.DMA,
    ],
)(_sc_body)


def kernel(content_all, content_all_mask, additional_bs, additional_bs_mask,
           content_prev_sep, pos_tags, wte, pos_table, token_weights, W_enc):
    absw, mask, idx = _tc_call(content_all, content_prev_sep, pos_tags,
                               additional_bs, W_enc, token_weights)
    content = _sc_call(wte, pos_table, absw, idx.reshape(-1), jnp.asarray(_DST))
    return content.reshape(B, S_OUT, D), mask


# speedup vs baseline: 1.0431x; 1.0431x over previous
import functools

import jax
import jax.numpy as jnp
import numpy as np
from jax import lax
from jax.experimental import pallas as pl
from jax.experimental.pallas import tpu as pltpu
from jax.experimental.pallas import tpu_sc as plsc

B = 16
L = 200
LA = 10
D_IN = 1024
D = 128
P = 75
S_OUT = 1 + LA + 2 + L + P

_HALF = L // 2
_WLOAD = 104
_PSPLIT = 40
_ASPLIT = 8
_NROWS = _WLOAD + _PSPLIT + _ASPLIT
_ACHUNK = 16


def _dst_tables():
    base = np.arange(B, dtype=np.int32)[:, None] * S_OUT
    h0 = np.concatenate([[0], 13 + np.arange(_HALF), [112] * (_WLOAD - _HALF - 1)])
    h1 = np.concatenate([113 + np.arange(_HALF), [212] * (_WLOAD - _HALF)])
    pos = np.concatenate([213 + np.arange(P), [287] * (2 * _PSPLIT - P)])
    arow = np.concatenate([1 + np.arange(12), [12] * (2 * _ASPLIT - 12)])
    c0 = np.concatenate([h0, pos[:_PSPLIT], arow[:_ASPLIT]]).astype(np.int32)
    c1 = np.concatenate([h1, pos[_PSPLIT:], arow[_ASPLIT:]]).astype(np.int32)
    return np.concatenate([(c0[None, :] + base).reshape(-1),
                           (c1[None, :] + base).reshape(-1)])


_DST = _dst_tables()


def _tc_body(ca_ref, sep_ref, pt_ref, abs_ref, w_ref, tw_ref,
             absw_ref, mask_ref, idxw_ref, idxp_ref):
    a = abs_ref[...].reshape(B * LA, D_IN)
    proj = lax.dot_general(a, w_ref[...], (((1,), (0,)), ((), ())),
                           preferred_element_type=jnp.float32)
    tw = tw_ref[...]
    t0 = jnp.broadcast_to(tw[0][None, None, :], (B, 1, D))
    t1 = jnp.broadcast_to(tw[1][None, None, :], (B, 5, D))
    absw_ref[...] = jnp.concatenate([proj.reshape(B, LA, D), t0, t1], axis=1)
    mask_ref[...] = jnp.ones((B, S_OUT), jnp.float32)
    ca = ca_ref[...]
    pt = pt_ref[...]
    h0 = jnp.concatenate(
        [sep_ref[...][:, :1], ca[:, :_HALF],
         jnp.broadcast_to(ca[:, _HALF - 1:_HALF], (B, 128 - _HALF - 1))], axis=1)
    h1 = jnp.concatenate(
        [ca[:, _HALF:], jnp.broadcast_to(ca[:, L - 1:L], (B, 128 - _HALF))], axis=1)
    idxw_ref[...] = jnp.concatenate([h0, h1], axis=0)
    idxp_ref[...] = jnp.concatenate(
        [pt, jnp.broadcast_to(pt[:, P - 1:P], (B, 128 - P))], axis=1)


_tc_call = pl.pallas_call(
    _tc_body,
    out_shape=(
        jax.ShapeDtypeStruct((B, _ACHUNK, D), jnp.float32),
        jax.ShapeDtypeStruct((B, S_OUT), jnp.float32),
        jax.ShapeDtypeStruct((2 * B, 128), jnp.int32),
        jax.ShapeDtypeStruct((B, 128), jnp.int32),
    ),
)


def _sc_body(wte, pos_table, absw, idx_wte, pos_idx, dst, out,
             idx_v, pidx_v, dst_v, rows_v, sem0, sem1, sem2):
    c = lax.axis_index("c")
    s = lax.axis_index("s")
    b = s
    chunk = c * B + s

    a1 = pltpu.async_copy(absw.at[b, pl.ds(c * _ASPLIT, _ASPLIT)],
                          rows_v.at[pl.ds(_WLOAD + _PSPLIT, _ASPLIT)], sem2)
    l1 = pltpu.async_copy(idx_wte.at[pl.ds(chunk * 128, _WLOAD)], idx_v, sem0)
    l2 = pltpu.async_copy(pos_idx.at[pl.ds(b * 128 + c * _PSPLIT, _PSPLIT)], pidx_v, sem0)
    l3 = pltpu.async_copy(dst.at[pl.ds(chunk * _NROWS, _NROWS)], dst_v, sem1)
    l1.wait()
    l2.wait()
    g1 = pltpu.async_copy(wte.at[idx_v], rows_v.at[pl.ds(0, _WLOAD)], sem0)
    g2 = pltpu.async_copy(pos_table.at[pidx_v], rows_v.at[pl.ds(_WLOAD, _PSPLIT)], sem0)
    l3.wait()
    a1.wait()
    g1.wait()
    g2.wait()
    pltpu.async_copy(rows_v, out.at[dst_v], sem2).wait()


_sc_call = functools.partial(
    pl.kernel,
    out_type=jax.ShapeDtypeStruct((B * S_OUT, D), jnp.float32),
    mesh=plsc.VectorSubcoreMesh(core_axis_name="c", subcore_axis_name="s"),
    scratch_types=[
        pltpu.VMEM((_WLOAD,), jnp.int32),
        pltpu.VMEM((_PSPLIT,), jnp.int32),
        pltpu.VMEM((_NROWS,), jnp.int32),
        pltpu.VMEM((_NROWS, D), jnp.float32),
        pltpu.SemaphoreType.DMA,
        pltpu.SemaphoreType.DMA,
        pltpu.SemaphoreType.DMA,
    ],
)(_sc_body)


def kernel(content_all, content_all_mask, additional_bs, additional_bs_mask,
           content_prev_sep, pos_tags, wte, pos_table, token_weights, W_enc):
    absw, mask, idxw, idxp = _tc_call(content_all, content_prev_sep, pos_tags,
                                      additional_bs, W_enc, token_weights)
    content = _sc_call(wte, pos_table, absw, idxw.reshape(-1), idxp.reshape(-1),
                       jnp.asarray(_DST))
    return content.reshape(B, S_OUT, D), mask


# speedup vs baseline: 1.0594x; 1.0156x over previous
import functools

import jax
import jax.numpy as jnp
import numpy as np
from jax import lax
from jax.experimental import pallas as pl
from jax.experimental.pallas import tpu as pltpu
from jax.experimental.pallas import tpu_sc as plsc

B = 16
L = 200
LA = 10
D_IN = 1024
D = 128
P = 75
S_OUT = 1 + LA + 2 + L + P

_HALF = L // 2
_WLOAD = 104
_PSPLIT = 40
_ASPLIT = 8
_SEP = 16
_NROWS = _WLOAD + _PSPLIT + _ASPLIT + _SEP
_CROWS = 44


def _const_tables():
    base = np.arange(B, dtype=np.int32)[:, None]
    sep = (np.arange(B, dtype=np.int32) * S_OUT)
    h0 = np.concatenate([13 + np.arange(_HALF), [112] * (_WLOAD - _HALF)])
    h1 = np.concatenate([113 + np.arange(_HALF), [212] * (_WLOAD - _HALF)])
    pos = np.concatenate([213 + np.arange(P), [287] * (2 * _PSPLIT - P)])
    arow = np.concatenate([1 + np.arange(12), [12] * (2 * _ASPLIT - 12)])
    c0 = np.concatenate([h0, pos[:_PSPLIT], arow[:_ASPLIT]]).astype(np.int32)
    c1 = np.concatenate([h1, pos[_PSPLIT:], arow[_ASPLIT:]]).astype(np.int32)
    dst0 = np.concatenate([c0[None, :] + base * S_OUT,
                           np.broadcast_to(sep, (B, _SEP))], axis=1)
    dst1 = np.concatenate([c1[None, :] + base * S_OUT,
                           np.broadcast_to(sep, (B, _SEP))], axis=1)
    ab0 = 16 * np.arange(8, dtype=np.int32)[None, :] + base
    ab1 = np.concatenate([128 + base, 144 + base,
                          np.broadcast_to(np.array([160, 161, 161, 161, 161, 161],
                                                   np.int32), (B, 6))], axis=1)
    flat = np.concatenate([dst0.reshape(-1), dst1.reshape(-1),
                           ab0.reshape(-1), ab1.reshape(-1)]).astype(np.int32)
    assert flat.size == _CROWS * 128, flat.size
    return flat.reshape(_CROWS, 128)


_CDATA = _const_tables()
_ABOFF = 2 * B * _NROWS


def _tc_body(ca_ref, pt_ref, abs_ref, w_ref, tw_ref, cin_ref,
             projtw_ref, mask_ref, idxw_ref, idxp_ref, cdata_ref):
    a = abs_ref[...].reshape(B * LA, D_IN)
    proj = lax.dot_general(a, w_ref[...], (((1,), (0,)), ((), ())),
                           preferred_element_type=jnp.float32)
    tw = tw_ref[...]
    projtw_ref[...] = jnp.concatenate(
        [proj, tw[0:1], jnp.broadcast_to(tw[1:2], (15, D))], axis=0)
    mask_ref[...] = jnp.ones((B, S_OUT), jnp.float32)
    ca = ca_ref[...]
    h0 = jnp.concatenate(
        [ca[:, :_HALF],
         jnp.broadcast_to(ca[:, _HALF - 1:_HALF], (B, 128 - _HALF))], axis=1)
    h1 = jnp.concatenate(
        [ca[:, _HALF:], jnp.broadcast_to(ca[:, L - 1:L], (B, 128 - _HALF))], axis=1)
    idxw_ref[...] = jnp.concatenate([h0, h1], axis=0)
    pt = pt_ref[...]
    idxp_ref[...] = jnp.concatenate(
        [pt, jnp.broadcast_to(pt[:, P - 1:P], (B, 128 - P))], axis=1)
    cdata_ref[...] = cin_ref[...]


_tc_call = pl.pallas_call(
    _tc_body,
    out_shape=(
        jax.ShapeDtypeStruct((B * LA + 16, D), jnp.float32),
        jax.ShapeDtypeStruct((B, S_OUT), jnp.float32),
        jax.ShapeDtypeStruct((2 * B, 128), jnp.int32),
        jax.ShapeDtypeStruct((B, 128), jnp.int32),
        jax.ShapeDtypeStruct((_CROWS, 128), jnp.int32),
    ),
)


def _sc_body(wte, pos_table, projtw, idx_wte, pos_idx, sep_idx, cdata, out,
             idx_v, pidx_v, sep_v, abidx_v, dst_v, rows_v, sem0, sem1, sem2):
    c = lax.axis_index("c")
    s = lax.axis_index("s")
    b = s
    chunk = c * B + s

    l1 = pltpu.async_copy(idx_wte.at[pl.ds(chunk * 128, _WLOAD)], idx_v, sem0)
    l2 = pltpu.async_copy(pos_idx.at[pl.ds(b * 128 + c * _PSPLIT, _PSPLIT)], pidx_v, sem0)
    l3 = pltpu.async_copy(sep_idx, sep_v, sem0)
    l4 = pltpu.async_copy(cdata.at[pl.ds(_ABOFF + chunk * _ASPLIT, _ASPLIT)], abidx_v, sem1)
    l5 = pltpu.async_copy(cdata.at[pl.ds(chunk * _NROWS, _NROWS)], dst_v, sem1)
    l1.wait()
    l2.wait()
    l3.wait()
    l4.wait()
    g1 = pltpu.async_copy(wte.at[idx_v], rows_v.at[pl.ds(0, _WLOAD)], sem0)
    g2 = pltpu.async_copy(pos_table.at[pidx_v], rows_v.at[pl.ds(_WLOAD, _PSPLIT)], sem0)
    g3 = pltpu.async_copy(projtw.at[abidx_v],
                          rows_v.at[pl.ds(_WLOAD + _PSPLIT, _ASPLIT)], sem0)
    g4 = pltpu.async_copy(wte.at[sep_v],
                          rows_v.at[pl.ds(_WLOAD + _PSPLIT + _ASPLIT, _SEP)], sem0)
    l5.wait()
    g1.wait()
    g2.wait()
    g3.wait()
    g4.wait()
    pltpu.async_copy(rows_v, out.at[dst_v], sem2).wait()


_sc_call = functools.partial(
    pl.kernel,
    out_type=jax.ShapeDtypeStruct((B * S_OUT, D), jnp.float32),
    mesh=plsc.VectorSubcoreMesh(core_axis_name="c", subcore_axis_name="s"),
    scratch_types=[
        pltpu.VMEM((_WLOAD,), jnp.int32),
        pltpu.VMEM((_PSPLIT,), jnp.int32),
        pltpu.VMEM((_SEP,), jnp.int32),
        pltpu.VMEM((_ASPLIT,), jnp.int32),
        pltpu.VMEM((_NROWS,), jnp.int32),
        pltpu.VMEM((_NROWS, D), jnp.float32),
        pltpu.SemaphoreType.DMA,
        pltpu.SemaphoreType.DMA,
        pltpu.SemaphoreType.DMA,
    ],
)(_sc_body)


def kernel(content_all, content_all_mask, additional_bs, additional_bs_mask,
           content_prev_sep, pos_tags, wte, pos_table, token_weights, W_enc):
    abs_t = jnp.transpose(additional_bs, (1, 0, 2))
    projtw, mask, idxw, idxp, cdata = _tc_call(content_all, pos_tags, abs_t,
                                               W_enc, token_weights,
                                               jnp.asarray(_CDATA))
    sep0 = content_prev_sep[:, 0]
    content = _sc_call(wte, pos_table, projtw, idxw.reshape(-1),
                       idxp.reshape(-1), sep0, cdata.reshape(-1))
    return content.reshape(B, S_OUT, D), mask


# speedup vs baseline: 1.0917x; 1.0305x over previous
import functools

import jax
import jax.numpy as jnp
import numpy as np
from jax import lax
from jax.experimental import pallas as pl
from jax.experimental.pallas import tpu as pltpu
from jax.experimental.pallas import tpu_sc as plsc

B = 16
L = 200
LA = 10
D_IN = 1024
D = 128
P = 75
S_OUT = 1 + LA + 2 + L + P

_HALF = L // 2
_WLOAD = 104
_PSPLIT = 40
_ASPLIT = 8
_NROWS = _WLOAD + _PSPLIT + _ASPLIT
_CROWS = 40


def _const_tables():
    base = np.arange(B, dtype=np.int32)[:, None]
    h0 = np.concatenate([[0], 13 + np.arange(_HALF), [112] * (_WLOAD - _HALF - 1)])
    h1 = np.concatenate([113 + np.arange(_HALF), [212] * (_WLOAD - _HALF)])
    pos = np.concatenate([213 + np.arange(P), [287] * (2 * _PSPLIT - P)])
    arow = np.concatenate([1 + np.arange(12), [12] * (2 * _ASPLIT - 12)])
    c0 = np.concatenate([h0, pos[:_PSPLIT], arow[:_ASPLIT]]).astype(np.int32)
    c1 = np.concatenate([h1, pos[_PSPLIT:], arow[_ASPLIT:]]).astype(np.int32)
    dst0 = c0[None, :] + base * S_OUT
    dst1 = c1[None, :] + base * S_OUT
    ab0 = 16 * np.arange(8, dtype=np.int32)[None, :] + base
    ab1 = np.concatenate([128 + base, 144 + base,
                          np.broadcast_to(np.array([160, 161, 161, 161, 161, 161],
                                                   np.int32), (B, 6))], axis=1)
    flat = np.concatenate([dst0.reshape(-1), dst1.reshape(-1),
                           ab0.reshape(-1), ab1.reshape(-1)]).astype(np.int32)
    assert flat.size == _CROWS * 128, flat.size
    return flat.reshape(_CROWS, 128)


_CDATA = _const_tables()
_ABOFF = 2 * B * _NROWS


def _tc_body(ca_ref, pt_ref, abs_ref, w_ref, tw_ref, cin_ref,
             projtw_ref, mask_ref, idxw_ref, idxp_ref, cdata_ref):
    a = abs_ref[...].reshape(B * LA, D_IN)
    proj = lax.dot_general(a, w_ref[...], (((1,), (0,)), ((), ())),
                           preferred_element_type=jnp.float32)
    tw = tw_ref[...]
    projtw_ref[...] = jnp.concatenate(
        [proj, tw[0:1], jnp.broadcast_to(tw[1:2], (15, D))], axis=0)
    mask_ref[...] = jnp.ones((B, S_OUT), jnp.float32)
    ca = ca_ref[...]
    h0 = jnp.concatenate(
        [ca[:, :1], ca[:, :_HALF],
         jnp.broadcast_to(ca[:, _HALF - 1:_HALF], (B, 128 - _HALF - 1))], axis=1)
    h1 = jnp.concatenate(
        [ca[:, _HALF:], jnp.broadcast_to(ca[:, L - 1:L], (B, 128 - _HALF))], axis=1)
    idxw_ref[...] = jnp.concatenate([h0, h1], axis=0)
    pt = pt_ref[...]
    idxp_ref[...] = jnp.concatenate(
        [pt, jnp.broadcast_to(pt[:, P - 1:P], (B, 128 - P))], axis=1)
    cdata_ref[...] = cin_ref[...]


_tc_call = pl.pallas_call(
    _tc_body,
    out_shape=(
        jax.ShapeDtypeStruct((B * LA + 16, D), jnp.float32),
        jax.ShapeDtypeStruct((B, S_OUT), jnp.float32),
        jax.ShapeDtypeStruct((2 * B, 128), jnp.int32),
        jax.ShapeDtypeStruct((B, 128), jnp.int32),
        jax.ShapeDtypeStruct((_CROWS, 128), jnp.int32),
    ),
)


def _sc_body(wte, pos_table, projtw, idx_wte, pos_idx, sep_idx, cdata, out,
             idx_v, pidx_v, sep_v, abidx_v, dst_v, rows_v, sem0, sem1, sem2):
    c = lax.axis_index("c")
    s = lax.axis_index("s")
    b = s
    chunk = c * B + s

    l1 = pltpu.async_copy(idx_wte.at[pl.ds(chunk * 128, _WLOAD)], idx_v, sem0)
    l2 = pltpu.async_copy(pos_idx.at[pl.ds(b * 128 + c * _PSPLIT, _PSPLIT)], pidx_v, sem0)
    l3 = pltpu.async_copy(sep_idx.at[pl.ds(b * 16, 16)], sep_v, sem0)
    l4 = pltpu.async_copy(cdata.at[pl.ds(_ABOFF + chunk * _ASPLIT, _ASPLIT)], abidx_v, sem1)
    l5 = pltpu.async_copy(cdata.at[pl.ds(chunk * _NROWS, _NROWS)], dst_v, sem1)
    l1.wait()
    l2.wait()
    l3.wait()
    l4.wait()
    l5.wait()

    @pl.when(c == 0)
    def _():
        lanes = lax.iota(jnp.int32, 16)
        idx_v[pl.ds(0, 16)] = jnp.where(lanes == 0, sep_v[...],
                                        idx_v[pl.ds(0, 16)])

    g1 = pltpu.async_copy(wte.at[idx_v], rows_v.at[pl.ds(0, _WLOAD)], sem0)
    g2 = pltpu.async_copy(pos_table.at[pidx_v], rows_v.at[pl.ds(_WLOAD, _PSPLIT)], sem0)
    g3 = pltpu.async_copy(projtw.at[abidx_v],
                          rows_v.at[pl.ds(_WLOAD + _PSPLIT, _ASPLIT)], sem0)
    g1.wait()
    g2.wait()
    g3.wait()
    pltpu.async_copy(rows_v, out.at[dst_v], sem2).wait()


_sc_call = functools.partial(
    pl.kernel,
    out_type=jax.ShapeDtypeStruct((B * S_OUT, D), jnp.float32),
    mesh=plsc.VectorSubcoreMesh(core_axis_name="c", subcore_axis_name="s"),
    scratch_types=[
        pltpu.VMEM((_WLOAD,), jnp.int32),
        pltpu.VMEM((_PSPLIT,), jnp.int32),
        pltpu.VMEM((16,), jnp.int32),
        pltpu.VMEM((_ASPLIT,), jnp.int32),
        pltpu.VMEM((_NROWS,), jnp.int32),
        pltpu.VMEM((_NROWS, D), jnp.float32),
        pltpu.SemaphoreType.DMA,
        pltpu.SemaphoreType.DMA,
        pltpu.SemaphoreType.DMA,
    ],
)(_sc_body)


def kernel(content_all, content_all_mask, additional_bs, additional_bs_mask,
           content_prev_sep, pos_tags, wte, pos_table, token_weights, W_enc):
    abs_t = jnp.transpose(additional_bs, (1, 0, 2))
    projtw, mask, idxw, idxp, cdata = _tc_call(content_all, pos_tags, abs_t,
                                               W_enc, token_weights,
                                               jnp.asarray(_CDATA))
    sep0 = jnp.repeat(content_prev_sep[:, 0], 16)
    content = _sc_call(wte, pos_table, projtw, idxw.reshape(-1),
                       idxp.reshape(-1), sep0, cdata.reshape(-1))
    return content.reshape(B, S_OUT, D), mask
